# baseline (speedup 1.0000x reference)
import numpy as np
import jax
import jax.numpy as jnp
from jax.experimental import pallas as pl

B = 4
C = 32
NS = 2048
KNN = 32
NH = 4
DM = 8
HD = C * DM
QB = 512
TW = HD + C
BISECT_CAP = 34


def _elu_fm(x):
    return jnp.where(x > 0, x + 1.0, jnp.exp(jnp.minimum(x, 0.0)))


def _np_consts():
    i = np.arange(C)[:, None]
    j = np.arange(HD)[None, :]
    rk = (i == j // DM).astype(np.float32)
    rv = (i == (j // (DM * DM)) * DM + j % DM).astype(np.float32)
    ih = np.arange(C)[:, None]
    jh = np.arange(C)[None, :]
    hb = (ih // DM == jh // DM).astype(np.float32)
    i2 = np.arange(HD)[:, None]
    j2 = np.arange(C)[None, :]
    s2 = ((i2 // (DM * DM) == j2 // DM)
          & (i2 % DM == j2 % DM)).astype(np.float32)
    return rk, rv, hb, s2


def _stage1(xt_ref, xyz_ref, w1_ref, b1_ref, w2_ref, b2_ref,
            qw_ref, kw_ref, vw_ref, rk_ref, rv_ref, t_ref, q_ref, na_ref):
    xt = xt_ref[...]
    xyz = xyz_ref[...]
    f32 = jnp.float32
    ones = jnp.ones((1, C), f32)
    d = lambda a, b: jax.lax.dot_general(a, b, (((1,), (1,)), ((), ())),
                                         preferred_element_type=f32)
    s = xt * xt
    s_hi = s.astype(jnp.bfloat16).astype(f32)
    r1 = s - s_hi
    s_lo = r1.astype(jnp.bfloat16).astype(f32)
    na_ref[0] = d(ones, s_hi) + d(ones, s_lo) + d(ones, r1 - s_lo)
    pos = jnp.maximum(xyz @ w1_ref[...] + b1_ref[...], 0.0)
    pos = pos @ w2_ref[...] + b2_ref[...]
    g = xt + pos
    q = _elu_fm(g @ qw_ref[...])
    kf = _elu_fm(g @ kw_ref[...])
    vals = (g @ vw_ref[...]) * (1.0 / KNN)
    w = (kf @ rk_ref[...]) * (vals @ rv_ref[...])
    t_ref[:, :HD] = w.astype(jnp.bfloat16)
    t_ref[:, HD:TW] = kf.astype(jnp.bfloat16)
    q_ref[...] = q


def _stage2(xq_ref, xa_ref, t_ref, qb_ref, na_ref, rk_ref, hb_ref, s2_ref,
            mw_ref, w1_ref, w2_ref,
            l1g_ref, l1b_ref, l2g_ref, l2b_ref, out_ref):
    xq = xq_ref[...]
    xa = xa_ref[...]
    f32 = jnp.float32
    ip = jax.lax.dot_general(xq.astype(jnp.bfloat16), xa.astype(jnp.bfloat16),
                             (((1,), (1,)), ((), ())),
                             preferred_element_type=f32)
    nq = jnp.sum(xq * xq, axis=1, keepdims=True)
    na = na_ref[0]
    pd = 2.0 * ip - nq - na

    lo = jnp.min(pd, axis=1, keepdims=True)
    hi = jnp.max(pd, axis=1, keepdims=True)
    cl = jnp.full((QB, 1), NS, jnp.int32)

    pdb = pd.astype(jnp.bfloat16)

    def cstep(c, _):
        lo, hi = c
        midb = (0.5 * (lo + hi)).astype(jnp.bfloat16)
        cnt = jnp.sum((pdb >= midb).astype(jnp.int32), axis=1, keepdims=True)
        ge = cnt >= KNN
        midf = midb.astype(jnp.float32)
        return (jnp.where(ge, midf, lo), jnp.where(ge, hi, midf)), None

    (lo, hi), _ = jax.lax.scan(cstep, (lo, hi), None, length=8)
    u = 2.0 ** -8
    lo = lo - u * jnp.abs(lo) - 1e-5
    hi = hi + u * jnp.abs(hi) + 1e-5

    def cond(carry):
        it, _, _, cl = carry
        return jnp.logical_and(it < BISECT_CAP, jnp.any(cl != KNN))

    def step(lo, hi, cl):
        mid = 0.5 * (lo + hi)
        cnt = jnp.sum((pd >= mid).astype(jnp.int32), axis=1, keepdims=True)
        ge = cnt >= KNN
        return (jnp.where(ge, mid, lo), jnp.where(ge, hi, mid),
                jnp.where(ge, cnt, cl))

    def body(carry):
        it, lo, hi, cl = carry
        lo, hi, cl = step(*step(lo, hi, cl))
        return (it + 2, lo, hi, cl)

    _, lo, _, _ = jax.lax.while_loop(cond, body, (0, lo, hi, cl))
    mask = (pd >= lo).astype(jnp.bfloat16)

    o = jnp.dot(mask, t_ref[...], preferred_element_type=f32)
    kv = o[:, :HD]
    s1 = o[:, HD:TW]
    qv = qb_ref[...]

    denom = (qv * s1) @ hb_ref[...]
    zf = 1.0 / (denom + 1e-6)
    prod = (qv @ rk_ref[...]) * kv
    msg = (prod @ s2_ref[...]) * zf * float(KNN)

    msg = msg @ mw_ref[...]
    m = jnp.mean(msg, axis=1, keepdims=True)
    v = jnp.mean((msg - m) ** 2, axis=1, keepdims=True)
    msg = (msg - m) * jax.lax.rsqrt(v + 1e-5) * l1g_ref[...] + l1b_ref[...]

    h = jnp.concatenate([xq, msg], axis=1)
    h = jnp.maximum(h @ w1_ref[...], 0.0) @ w2_ref[...]
    m = jnp.mean(h, axis=1, keepdims=True)
    v = jnp.mean((h - m) ** 2, axis=1, keepdims=True)
    h = (h - m) * jax.lax.rsqrt(v + 1e-5) * l2g_ref[...] + l2b_ref[...]

    out_ref[...] = xq + h


def kernel(search_feat, search_xyz, pos_w1, pos_b1, pos_w2, pos_b2,
           q_w, k_w, v_w, merge_w, mlp_w1, mlp_w2,
           ln1_g, ln1_b, ln2_g, ln2_b):
    f32 = jnp.float32
    xt = jnp.transpose(search_feat, (0, 2, 1)).reshape(B * NS, C)
    xyz = search_xyz.reshape(B * NS, 3)
    xyz = jnp.pad(xyz, ((0, 0), (0, 5)))
    w1p = jnp.pad(pos_w1, ((0, 5), (0, 0)))

    rk_np, rv_np, hb_np, s2_np = _np_consts()
    rk = jnp.asarray(rk_np)
    rv = jnp.asarray(rv_np)
    hb = jnp.asarray(hb_np)
    s2 = jnp.asarray(s2_np)

    b1 = pos_b1.reshape(1, 32)
    b2 = pos_b2.reshape(1, 32)
    l1g = ln1_g.reshape(1, C)
    l1b = ln1_b.reshape(1, C)
    l2g = ln2_g.reshape(1, C)
    l2b = ln2_b.reshape(1, C)

    full = lambda shape: pl.BlockSpec(shape, lambda i: tuple(0 for _ in shape))

    t_tab, q_tab, na_row = pl.pallas_call(
        _stage1,
        grid=(B,),
        in_specs=[
            pl.BlockSpec((NS, C), lambda i: (i, 0)),
            pl.BlockSpec((NS, 8), lambda i: (i, 0)),
            full((8, 32)), full((1, 32)), full((32, 32)), full((1, 32)),
            full((C, C)), full((C, C)), full((C, C)),
            full((C, HD)), full((C, HD)),
        ],
        out_specs=[
            pl.BlockSpec((NS, TW), lambda i: (i, 0)),
            pl.BlockSpec((NS, C), lambda i: (i, 0)),
            pl.BlockSpec((1, 1, NS), lambda i: (i, 0, 0)),
        ],
        out_shape=[
            jax.ShapeDtypeStruct((B * NS, TW), jnp.bfloat16),
            jax.ShapeDtypeStruct((B * NS, C), f32),
            jax.ShapeDtypeStruct((B, 1, NS), f32),
        ],
    )(xt, xyz, w1p, b1, pos_w2, b2, q_w, k_w, v_w, rk, rv)

    nblk = (B * NS) // QB
    out = pl.pallas_call(
        _stage2,
        grid=(nblk,),
        in_specs=[
            pl.BlockSpec((QB, C), lambda i: (i, 0)),
            pl.BlockSpec((NS, C), lambda i: (i // (NS // QB), 0)),
            pl.BlockSpec((NS, TW), lambda i: (i // (NS // QB), 0)),
            pl.BlockSpec((QB, C), lambda i: (i, 0)),
            pl.BlockSpec((1, 1, NS), lambda i: (i // (NS // QB), 0, 0)),
            full((C, HD)), full((C, C)), full((HD, C)),
            full((C, C)), full((2 * C, 2 * C)), full((2 * C, C)),
            full((1, C)), full((1, C)), full((1, C)), full((1, C)),
        ],
        out_specs=pl.BlockSpec((QB, C), lambda i: (i, 0)),
        out_shape=jax.ShapeDtypeStruct((B * NS, C), f32),
    )(xt, xt, t_tab, q_tab, na_row, rk, hb, s2, merge_w, mlp_w1, mlp_w2,
      l1g, l1b, l2g, l2b)

    return jnp.transpose(out.reshape(B, NS, C), (0, 2, 1))


# speedup vs baseline: 1.3953x; 1.3953x over previous
import numpy as np
import jax
import jax.numpy as jnp
from jax.experimental import pallas as pl

B = 4
C = 32
NS = 2048
KNN = 32
NH = 4
DM = 8
HD = C * DM
QB = 1024
TW = HD + C
BISECT_CAP = 34


def _elu_fm(x):
    return jnp.where(x > 0, x + 1.0, jnp.exp(jnp.minimum(x, 0.0)))


def _np_consts():
    i = np.arange(C)[:, None]
    j = np.arange(HD)[None, :]
    rk = (i == j // DM).astype(np.float32)
    rv = (i == (j // (DM * DM)) * DM + j % DM).astype(np.float32)
    ih = np.arange(C)[:, None]
    jh = np.arange(C)[None, :]
    hb = (ih // DM == jh // DM).astype(np.float32)
    i2 = np.arange(HD)[:, None]
    j2 = np.arange(C)[None, :]
    s2 = ((i2 // (DM * DM) == j2 // DM)
          & (i2 % DM == j2 % DM)).astype(np.float32)
    return rk, rv, hb, s2


def _stage1(xt_ref, xyz_ref, w1_ref, b1_ref, w2_ref, b2_ref,
            qw_ref, kw_ref, vw_ref, rk_ref, rv_ref, t_ref, q_ref, na_ref):
    xt = xt_ref[...]
    xyz = xyz_ref[...]
    f32 = jnp.float32
    ones = jnp.ones((1, C), f32)
    d = lambda a, b: jax.lax.dot_general(a, b, (((1,), (1,)), ((), ())),
                                         preferred_element_type=f32)
    s = xt * xt
    s_hi = s.astype(jnp.bfloat16).astype(f32)
    r1 = s - s_hi
    s_lo = r1.astype(jnp.bfloat16).astype(f32)
    na_ref[0] = d(ones, s_hi) + d(ones, s_lo) + d(ones, r1 - s_lo)
    pos = jnp.maximum(xyz @ w1_ref[...] + b1_ref[...], 0.0)
    pos = pos @ w2_ref[...] + b2_ref[...]
    g = xt + pos
    q = _elu_fm(g @ qw_ref[...])
    kf = _elu_fm(g @ kw_ref[...])
    vals = (g @ vw_ref[...]) * (1.0 / KNN)
    w = (kf @ rk_ref[...]) * (vals @ rv_ref[...])
    t_ref[:, :HD] = w.astype(jnp.bfloat16)
    t_ref[:, HD:TW] = kf.astype(jnp.bfloat16)
    q_ref[...] = q


def _stage2(xq_ref, xa_ref, t_ref, qb_ref, na_ref, rk_ref, hb_ref, s2_ref,
            mw_ref, w1_ref, w2_ref,
            l1g_ref, l1b_ref, l2g_ref, l2b_ref, out_ref):
    xq = xq_ref[...]
    xa = xa_ref[...]
    f32 = jnp.float32
    ip = jax.lax.dot_general(xq.astype(jnp.bfloat16), xa.astype(jnp.bfloat16),
                             (((1,), (1,)), ((), ())),
                             preferred_element_type=f32)
    nq = jnp.sum(xq * xq, axis=1, keepdims=True)
    na = na_ref[0]
    pd = 2.0 * ip - nq - na

    lo = jnp.min(pd, axis=1, keepdims=True)
    hi = jnp.max(pd, axis=1, keepdims=True)
    cl = jnp.full((QB, 1), NS, jnp.int32)

    def cond(carry):
        it, _, _, cl = carry
        return jnp.logical_and(it < BISECT_CAP,
                               jnp.any(jnp.logical_or(cl < KNN, cl > KNN + 1)))

    def step(lo, hi, cl):
        mid = 0.5 * (lo + hi)
        cnt = jnp.sum((pd >= mid).astype(jnp.int32), axis=1, keepdims=True)
        ge = cnt >= KNN
        return (jnp.where(ge, mid, lo), jnp.where(ge, hi, mid),
                jnp.where(ge, cnt, cl))

    def body(carry):
        it, lo, hi, cl = carry
        lo, hi, cl = step(*step(lo, hi, cl))
        return (it + 2, lo, hi, cl)

    _, lo, _, _ = jax.lax.while_loop(cond, body, (0, lo, hi, cl))
    sel = pd >= lo
    c = jnp.sum(sel.astype(jnp.int32), axis=1, keepdims=True)
    m33 = jnp.min(jnp.where(sel, pd, jnp.float32(jnp.inf)), axis=1,
                  keepdims=True)
    drop = jnp.logical_and(c == KNN + 1, pd == m33)
    mask = jnp.logical_and(sel, jnp.logical_not(drop)).astype(jnp.bfloat16)

    o = jnp.dot(mask, t_ref[...], preferred_element_type=f32)
    kv = o[:, :HD]
    s1 = o[:, HD:TW]
    qv = qb_ref[...]

    denom = (qv * s1) @ hb_ref[...]
    zf = 1.0 / (denom + 1e-6)
    prod = (qv @ rk_ref[...]) * kv
    msg = (prod @ s2_ref[...]) * zf * float(KNN)

    msg = msg @ mw_ref[...]
    m = jnp.mean(msg, axis=1, keepdims=True)
    v = jnp.mean((msg - m) ** 2, axis=1, keepdims=True)
    msg = (msg - m) * jax.lax.rsqrt(v + 1e-5) * l1g_ref[...] + l1b_ref[...]

    h = jnp.concatenate([xq, msg], axis=1)
    h = jnp.maximum(h @ w1_ref[...], 0.0) @ w2_ref[...]
    m = jnp.mean(h, axis=1, keepdims=True)
    v = jnp.mean((h - m) ** 2, axis=1, keepdims=True)
    h = (h - m) * jax.lax.rsqrt(v + 1e-5) * l2g_ref[...] + l2b_ref[...]

    out_ref[...] = xq + h


def kernel(search_feat, search_xyz, pos_w1, pos_b1, pos_w2, pos_b2,
           q_w, k_w, v_w, merge_w, mlp_w1, mlp_w2,
           ln1_g, ln1_b, ln2_g, ln2_b):
    f32 = jnp.float32
    xt = jnp.transpose(search_feat, (0, 2, 1)).reshape(B * NS, C)
    xyz = search_xyz.reshape(B * NS, 3)
    xyz = jnp.pad(xyz, ((0, 0), (0, 5)))
    w1p = jnp.pad(pos_w1, ((0, 5), (0, 0)))

    rk_np, rv_np, hb_np, s2_np = _np_consts()
    rk = jnp.asarray(rk_np)
    rv = jnp.asarray(rv_np)
    hb = jnp.asarray(hb_np)
    s2 = jnp.asarray(s2_np)

    b1 = pos_b1.reshape(1, 32)
    b2 = pos_b2.reshape(1, 32)
    l1g = ln1_g.reshape(1, C)
    l1b = ln1_b.reshape(1, C)
    l2g = ln2_g.reshape(1, C)
    l2b = ln2_b.reshape(1, C)

    full = lambda shape: pl.BlockSpec(shape, lambda i: tuple(0 for _ in shape))

    t_tab, q_tab, na_row = pl.pallas_call(
        _stage1,
        grid=(B,),
        in_specs=[
            pl.BlockSpec((NS, C), lambda i: (i, 0)),
            pl.BlockSpec((NS, 8), lambda i: (i, 0)),
            full((8, 32)), full((1, 32)), full((32, 32)), full((1, 32)),
            full((C, C)), full((C, C)), full((C, C)),
            full((C, HD)), full((C, HD)),
        ],
        out_specs=[
            pl.BlockSpec((NS, TW), lambda i: (i, 0)),
            pl.BlockSpec((NS, C), lambda i: (i, 0)),
            pl.BlockSpec((1, 1, NS), lambda i: (i, 0, 0)),
        ],
        out_shape=[
            jax.ShapeDtypeStruct((B * NS, TW), jnp.bfloat16),
            jax.ShapeDtypeStruct((B * NS, C), f32),
            jax.ShapeDtypeStruct((B, 1, NS), f32),
        ],
    )(xt, xyz, w1p, b1, pos_w2, b2, q_w, k_w, v_w, rk, rv)

    nblk = (B * NS) // QB
    out = pl.pallas_call(
        _stage2,
        grid=(nblk,),
        in_specs=[
            pl.BlockSpec((QB, C), lambda i: (i, 0)),
            pl.BlockSpec((NS, C), lambda i: (i // (NS // QB), 0)),
            pl.BlockSpec((NS, TW), lambda i: (i // (NS // QB), 0)),
            pl.BlockSpec((QB, C), lambda i: (i, 0)),
            pl.BlockSpec((1, 1, NS), lambda i: (i // (NS // QB), 0, 0)),
            full((C, HD)), full((C, C)), full((HD, C)),
            full((C, C)), full((2 * C, 2 * C)), full((2 * C, C)),
            full((1, C)), full((1, C)), full((1, C)), full((1, C)),
        ],
        out_specs=pl.BlockSpec((QB, C), lambda i: (i, 0)),
        out_shape=jax.ShapeDtypeStruct((B * NS, C), f32),
    )(xt, xt, t_tab, q_tab, na_row, rk, hb, s2, merge_w, mlp_w1, mlp_w2,
      l1g, l1b, l2g, l2b)

    return jnp.transpose(out.reshape(B, NS, C), (0, 2, 1))


# speedup vs baseline: 1.4067x; 1.0082x over previous
import numpy as np
import jax
import jax.numpy as jnp
from jax.experimental import pallas as pl

B = 4
C = 32
NS = 2048
KNN = 32
NH = 4
DM = 8
HD = C * DM
QB = 512
TW = HD + C
BISECT_CAP = 34


def _elu_fm(x):
    return jnp.where(x > 0, x + 1.0, jnp.exp(jnp.minimum(x, 0.0)))


def _np_consts():
    i = np.arange(C)[:, None]
    j = np.arange(HD)[None, :]
    rk = (i == j // DM).astype(np.float32)
    rv = (i == (j // (DM * DM)) * DM + j % DM).astype(np.float32)
    ih = np.arange(C)[:, None]
    jh = np.arange(C)[None, :]
    hb = (ih // DM == jh // DM).astype(np.float32)
    i2 = np.arange(HD)[:, None]
    j2 = np.arange(C)[None, :]
    s2 = ((i2 // (DM * DM) == j2 // DM)
          & (i2 % DM == j2 % DM)).astype(np.float32)
    return rk, rv, hb, s2


def _stage1(xt_ref, xyz_ref, w1_ref, b1_ref, w2_ref, b2_ref,
            qw_ref, kw_ref, vw_ref, rk_ref, rv_ref, t_ref, q_ref, na_ref):
    xt = xt_ref[...]
    xyz = xyz_ref[...]
    f32 = jnp.float32
    ones = jnp.ones((1, C), f32)
    d = lambda a, b: jax.lax.dot_general(a, b, (((1,), (1,)), ((), ())),
                                         preferred_element_type=f32)
    s = xt * xt
    s_hi = s.astype(jnp.bfloat16).astype(f32)
    r1 = s - s_hi
    s_lo = r1.astype(jnp.bfloat16).astype(f32)
    na_ref[0] = d(ones, s_hi) + d(ones, s_lo) + d(ones, r1 - s_lo)
    pos = jnp.maximum(xyz @ w1_ref[...] + b1_ref[...], 0.0)
    pos = pos @ w2_ref[...] + b2_ref[...]
    g = xt + pos
    q = _elu_fm(g @ qw_ref[...])
    kf = _elu_fm(g @ kw_ref[...])
    vals = (g @ vw_ref[...]) * (1.0 / KNN)
    w = (kf @ rk_ref[...]) * (vals @ rv_ref[...])
    t_ref[:, :HD] = w.astype(jnp.bfloat16)
    t_ref[:, HD:TW] = kf.astype(jnp.bfloat16)
    q_ref[...] = q


def _stage2(xq_ref, xa_ref, t_ref, qb_ref, na_ref, rk_ref, hb_ref, s2_ref,
            mw_ref, w1_ref, w2_ref,
            l1g_ref, l1b_ref, l2g_ref, l2b_ref, out_ref):
    xq = xq_ref[...]
    xa = xa_ref[...]
    f32 = jnp.float32
    ip = jax.lax.dot_general(xq.astype(jnp.bfloat16), xa.astype(jnp.bfloat16),
                             (((1,), (1,)), ((), ())),
                             preferred_element_type=f32)
    nq = jnp.sum(xq * xq, axis=1, keepdims=True)
    na = na_ref[0]
    pd = 2.0 * ip - nq - na

    lo = jnp.min(pd, axis=1, keepdims=True)
    hi = jnp.max(pd, axis=1, keepdims=True)
    cl = jnp.full((QB, 1), NS, jnp.int32)

    def cond(carry):
        it, _, _, cl = carry
        return jnp.logical_and(it < BISECT_CAP,
                               jnp.any(jnp.logical_or(cl < KNN, cl > KNN + 1)))

    def step(lo, hi, cl):
        mid = 0.5 * (lo + hi)
        cnt = jnp.sum((pd >= mid).astype(jnp.int32), axis=1, keepdims=True)
        ge = cnt >= KNN
        return (jnp.where(ge, mid, lo), jnp.where(ge, hi, mid),
                jnp.where(ge, cnt, cl))

    def body(carry):
        it, lo, hi, cl = carry
        lo, hi, cl = step(*step(lo, hi, cl))
        return (it + 2, lo, hi, cl)

    _, lo, _, _ = jax.lax.while_loop(cond, body, (0, lo, hi, cl))
    sel = pd >= lo
    c = jnp.sum(sel.astype(jnp.int32), axis=1, keepdims=True)
    m33 = jnp.min(jnp.where(sel, pd, jnp.float32(jnp.inf)), axis=1,
                  keepdims=True)
    drop = jnp.logical_and(c == KNN + 1, pd == m33)
    mask = jnp.logical_and(sel, jnp.logical_not(drop)).astype(jnp.bfloat16)

    o = jnp.dot(mask, t_ref[...], preferred_element_type=f32)
    kv = o[:, :HD]
    s1 = o[:, HD:TW]
    qv = qb_ref[...]

    denom = (qv * s1) @ hb_ref[...]
    zf = 1.0 / (denom + 1e-6)
    prod = (qv @ rk_ref[...]) * kv
    msg = (prod @ s2_ref[...]) * zf * float(KNN)

    msg = msg @ mw_ref[...]
    m = jnp.mean(msg, axis=1, keepdims=True)
    v = jnp.mean((msg - m) ** 2, axis=1, keepdims=True)
    msg = (msg - m) * jax.lax.rsqrt(v + 1e-5) * l1g_ref[...] + l1b_ref[...]

    h = jnp.concatenate([xq, msg], axis=1)
    h = jnp.maximum(h @ w1_ref[...], 0.0) @ w2_ref[...]
    m = jnp.mean(h, axis=1, keepdims=True)
    v = jnp.mean((h - m) ** 2, axis=1, keepdims=True)
    h = (h - m) * jax.lax.rsqrt(v + 1e-5) * l2g_ref[...] + l2b_ref[...]

    out_ref[...] = xq + h


def kernel(search_feat, search_xyz, pos_w1, pos_b1, pos_w2, pos_b2,
           q_w, k_w, v_w, merge_w, mlp_w1, mlp_w2,
           ln1_g, ln1_b, ln2_g, ln2_b):
    f32 = jnp.float32
    xt = jnp.transpose(search_feat, (0, 2, 1)).reshape(B * NS, C)
    xyz = search_xyz.reshape(B * NS, 3)
    xyz = jnp.pad(xyz, ((0, 0), (0, 5)))
    w1p = jnp.pad(pos_w1, ((0, 5), (0, 0)))

    rk_np, rv_np, hb_np, s2_np = _np_consts()
    rk = jnp.asarray(rk_np)
    rv = jnp.asarray(rv_np)
    hb = jnp.asarray(hb_np)
    s2 = jnp.asarray(s2_np)

    b1 = pos_b1.reshape(1, 32)
    b2 = pos_b2.reshape(1, 32)
    l1g = ln1_g.reshape(1, C)
    l1b = ln1_b.reshape(1, C)
    l2g = ln2_g.reshape(1, C)
    l2b = ln2_b.reshape(1, C)

    full = lambda shape: pl.BlockSpec(shape, lambda i: tuple(0 for _ in shape))

    t_tab, q_tab, na_row = pl.pallas_call(
        _stage1,
        grid=(B,),
        in_specs=[
            pl.BlockSpec((NS, C), lambda i: (i, 0)),
            pl.BlockSpec((NS, 8), lambda i: (i, 0)),
            full((8, 32)), full((1, 32)), full((32, 32)), full((1, 32)),
            full((C, C)), full((C, C)), full((C, C)),
            full((C, HD)), full((C, HD)),
        ],
        out_specs=[
            pl.BlockSpec((NS, TW), lambda i: (i, 0)),
            pl.BlockSpec((NS, C), lambda i: (i, 0)),
            pl.BlockSpec((1, 1, NS), lambda i: (i, 0, 0)),
        ],
        out_shape=[
            jax.ShapeDtypeStruct((B * NS, TW), jnp.bfloat16),
            jax.ShapeDtypeStruct((B * NS, C), f32),
            jax.ShapeDtypeStruct((B, 1, NS), f32),
        ],
    )(xt, xyz, w1p, b1, pos_w2, b2, q_w, k_w, v_w, rk, rv)

    nblk = (B * NS) // QB
    out = pl.pallas_call(
        _stage2,
        grid=(nblk,),
        in_specs=[
            pl.BlockSpec((QB, C), lambda i: (i, 0)),
            pl.BlockSpec((NS, C), lambda i: (i // (NS // QB), 0)),
            pl.BlockSpec((NS, TW), lambda i: (i // (NS // QB), 0)),
            pl.BlockSpec((QB, C), lambda i: (i, 0)),
            pl.BlockSpec((1, 1, NS), lambda i: (i // (NS // QB), 0, 0)),
            full((C, HD)), full((C, C)), full((HD, C)),
            full((C, C)), full((2 * C, 2 * C)), full((2 * C, C)),
            full((1, C)), full((1, C)), full((1, C)), full((1, C)),
        ],
        out_specs=pl.BlockSpec((QB, C), lambda i: (i, 0)),
        out_shape=jax.ShapeDtypeStruct((B * NS, C), f32),
    )(xt, xt, t_tab, q_tab, na_row, rk, hb, s2, merge_w, mlp_w1, mlp_w2,
      l1g, l1b, l2g, l2b)

    return jnp.transpose(out.reshape(B, NS, C), (0, 2, 1))
